# Initial kernel scaffold; baseline (speedup 1.0000x reference)
import jax
import jax.numpy as jnp
from jax.experimental import pallas as pl


def kernel(user_feat, repo_feat, edge_index_ur, pos_edge_index, neg_edge_index, W_user, b_user, W_repo, b_repo, W_h_ur, b_h_ur, W_h_ru, b_h_ru, W_o_ur, b_o_ur, W_o_ru, b_o_ru):
    raise NotImplementedError("write your pallas kernel here")


# speedup vs baseline: 5.8394x; 5.8394x over previous
import functools

import jax
import jax.numpy as jnp
from jax import lax
from jax.experimental import pallas as pl
from jax.experimental.pallas import tpu as pltpu
from jax.experimental.pallas import tpu_sc as plsc

N_SC = 2
N_TILES = 16
NW = N_SC * N_TILES
LANES = 128


def _cdiv(a, b):
    return (a + b - 1) // b


def _tc_embed_body(n_real, uf, rf, Wu, bu, Wr, br, Whur, Whru, degu, degr,
                   xu_out, xr_out):
    npad = uf.shape[0]
    rows = lax.broadcasted_iota(jnp.int32, (npad, 1), 0)
    mask = rows < n_real

    def side(feat, W1, b1, W2, deg2, out_ref):
        deg = jnp.max(deg2[0], axis=1, keepdims=True) + \
              jnp.max(deg2[1], axis=1, keepdims=True)
        norm = lax.rsqrt(jnp.maximum(deg, 1.0))
        h = jnp.dot(feat[...], W1[...], preferred_element_type=jnp.float32)
        h = h + b1[...]
        x = jnp.dot(h, W2[...], preferred_element_type=jnp.float32) * norm
        out_ref[...] = jnp.where(mask, x, 0.0)

    side(uf, Wu, bu, Whur, degu, xu_out)
    side(rf, Wr, br, Whru, degr, xr_out)


def _tc_mid_body(n_real, aggu, aggr, degu, degr, bhru, bhur, Wour, Woru,
                 yu_out, yr_out):
    npad = aggu.shape[1]
    rows = lax.broadcasted_iota(jnp.int32, (npad, 1), 0)
    mask = rows < n_real

    def side(agg2, deg2, b_in, W2, out_ref):
        deg = jnp.max(deg2[0], axis=1, keepdims=True) + \
              jnp.max(deg2[1], axis=1, keepdims=True)
        norm = lax.rsqrt(jnp.maximum(deg, 1.0))
        h1 = (agg2[0] + agg2[1]) * norm + b_in[...]
        y = jnp.dot(h1, W2[...], preferred_element_type=jnp.float32) * norm
        out_ref[...] = jnp.where(mask, y, 0.0)

    side(aggu, degu, bhru, Wour, yu_out)
    side(aggr, degr, bhur, Woru, yr_out)


def _tc_final_body(n_real, aggu, aggr, degu, degr, boru, bour,
                   nu_out, nr_out):
    npad = aggu.shape[1]
    rows = lax.broadcasted_iota(jnp.int32, (npad, 1), 0)
    mask = rows < n_real

    def side(agg2, deg2, b_in, out_ref):
        deg = jnp.max(deg2[0], axis=1, keepdims=True) + \
              jnp.max(deg2[1], axis=1, keepdims=True)
        norm = lax.rsqrt(jnp.maximum(deg, 1.0))
        o = (agg2[0] + agg2[1]) * norm + b_in[...]
        nrm = jnp.sqrt(jnp.sum(o * o, axis=-1, keepdims=True))
        out_ref[...] = jnp.where(mask, o / jnp.maximum(nrm, 1e-12), 0.0)

    side(aggu, degu, boru, nu_out)
    side(aggr, degr, bour, nr_out)


def _tc_call(body, out_shapes, *args):
    return pl.pallas_call(
        body,
        out_shape=[jax.ShapeDtypeStruct(s, jnp.float32) for s in out_shapes],
    )(*args)


def _make_mesh():
    return plsc.VectorSubcoreMesh(core_axis_name="c", subcore_axis_name="s",
                                  num_cores=N_SC, num_subcores=N_TILES)


def _make_deg_kernel(npad, erows_pad):
    rpt = erows_pad // NW
    spt = npad // N_TILES

    @functools.partial(
        pl.kernel,
        out_type=[jax.ShapeDtypeStruct((N_SC, npad, 16), jnp.float32),
                  jax.ShapeDtypeStruct((N_SC, npad, 16), jnp.float32)],
        mesh=_make_mesh(),
        compiler_params=pltpu.CompilerParams(use_tc_tiling_on_sc=False, needs_layout_passes=False),
        scratch_types=[
            pltpu.VMEM_SHARED((npad, 16), jnp.float32),
            pltpu.VMEM_SHARED((npad, 16), jnp.float32),
            pltpu.VMEM((rpt, LANES), jnp.int32),
            pltpu.VMEM((rpt, LANES), jnp.int32),
            pltpu.VMEM((LANES, 16), jnp.float32),
            pltpu.SemaphoreType.DMA((2,)),
            pltpu.SemaphoreType.DMA((2,)),
        ],
    )
    def deg_kernel(e3, z16, ones16, degu_out, degr_out,
                   sh_du, sh_dr, uidx, ridx, ones_v, sdu, sdr):
        c = lax.axis_index("c")
        s = lax.axis_index("s")
        wid = c * N_TILES + s
        off = s * spt
        pltpu.sync_copy(z16.at[pl.ds(off, spt)], sh_du.at[pl.ds(off, spt)])
        pltpu.sync_copy(z16.at[pl.ds(off, spt)], sh_dr.at[pl.ds(off, spt)])
        pltpu.sync_copy(e3.at[0, pl.ds(wid * rpt, rpt)], uidx)
        pltpu.sync_copy(e3.at[1, pl.ds(wid * rpt, rpt)], ridx)
        pltpu.sync_copy(ones16, ones_v)
        plsc.subcore_barrier()

        def wait_deg(p):
            pltpu.make_async_copy(ones_v, sh_du.at[uidx.at[0]],
                                  sdu.at[p]).wait()
            pltpu.make_async_copy(ones_v, sh_dr.at[ridx.at[0]],
                                  sdr.at[p]).wait()

        def body(jj, carry):
            j0 = 2 * jj
            j1 = j0 + 1

            @pl.when(jj > 0)
            def _():
                wait_deg(0)
                wait_deg(1)

            pltpu.async_copy(ones_v, sh_du.at[uidx.at[j0]], sdu.at[0],
                             add=True)
            pltpu.async_copy(ones_v, sh_dr.at[ridx.at[j0]], sdr.at[0],
                             add=True)
            pltpu.async_copy(ones_v, sh_du.at[uidx.at[j1]], sdu.at[1],
                             add=True)
            pltpu.async_copy(ones_v, sh_dr.at[ridx.at[j1]], sdr.at[1],
                             add=True)
            return carry

        lax.fori_loop(0, rpt // 2, body, 0)
        wait_deg(0)
        wait_deg(1)
        plsc.subcore_barrier()
        pltpu.sync_copy(sh_du.at[pl.ds(off, spt)],
                        degu_out.at[c, pl.ds(off, spt)])
        pltpu.sync_copy(sh_dr.at[pl.ds(off, spt)],
                        degr_out.at[c, pl.ds(off, spt)])

    return deg_kernel


def _make_spmm_kernel(npad, erows_pad, w):
    rpt = erows_pad // NW
    spt = npad // N_TILES

    @functools.partial(
        pl.kernel,
        out_type=[jax.ShapeDtypeStruct((N_SC, npad, w), jnp.float32),
                  jax.ShapeDtypeStruct((N_SC, npad, w), jnp.float32)],
        mesh=_make_mesh(),
        compiler_params=pltpu.CompilerParams(use_tc_tiling_on_sc=False, needs_layout_passes=False),
        scratch_types=[
            pltpu.VMEM_SHARED((npad, w), jnp.float32),
            pltpu.VMEM_SHARED((npad, w), jnp.float32),
            pltpu.VMEM((rpt, LANES), jnp.int32),
            pltpu.VMEM((rpt, LANES), jnp.int32),
            pltpu.VMEM((2, LANES, w), jnp.float32),
            pltpu.VMEM((2, LANES, w), jnp.float32),
            pltpu.SemaphoreType.DMA((2,)),
            pltpu.SemaphoreType.DMA((2,)),
            pltpu.SemaphoreType.DMA((2,)),
            pltpu.SemaphoreType.DMA((2,)),
        ],
    )
    def spmm_kernel(e3, xu, xr, zw, aggu_out, aggr_out,
                    sh_au, sh_ar, uidx, ridx, rowa, rowb,
                    gsem_a, gsem_b, ssem_a, ssem_b):
        c = lax.axis_index("c")
        s = lax.axis_index("s")
        wid = c * N_TILES + s
        off = s * spt
        pltpu.sync_copy(zw.at[pl.ds(off, spt)], sh_au.at[pl.ds(off, spt)])
        pltpu.sync_copy(zw.at[pl.ds(off, spt)], sh_ar.at[pl.ds(off, spt)])
        pltpu.sync_copy(e3.at[0, pl.ds(wid * rpt, rpt)], uidx)
        pltpu.sync_copy(e3.at[1, pl.ds(wid * rpt, rpt)], ridx)
        plsc.subcore_barrier()

        def start_gathers(j, p):
            pltpu.async_copy(xr.at[ridx.at[j]], rowa.at[p], gsem_a.at[p])
            pltpu.async_copy(xu.at[uidx.at[j]], rowb.at[p], gsem_b.at[p])

        def wait_gathers(p):
            pltpu.make_async_copy(xr.at[ridx.at[0]], rowa.at[p],
                                  gsem_a.at[p]).wait()
            pltpu.make_async_copy(xu.at[uidx.at[0]], rowb.at[p],
                                  gsem_b.at[p]).wait()

        def start_scatters(j, p):
            pltpu.async_copy(rowa.at[p], sh_au.at[uidx.at[j]], ssem_a.at[p],
                             add=True)
            pltpu.async_copy(rowb.at[p], sh_ar.at[ridx.at[j]], ssem_b.at[p],
                             add=True)

        def wait_scatters(p):
            pltpu.make_async_copy(rowa.at[p], sh_au.at[uidx.at[0]],
                                  ssem_a.at[p]).wait()
            pltpu.make_async_copy(rowb.at[p], sh_ar.at[ridx.at[0]],
                                  ssem_b.at[p]).wait()

        half = rpt // 2
        start_gathers(0, 0)

        def body(jj, carry):
            j0 = 2 * jj
            j1 = j0 + 1

            @pl.when(jj > 0)
            def _():
                wait_scatters(1)

            start_gathers(j1, 1)
            wait_gathers(0)
            start_scatters(j0, 0)
            wait_gathers(1)
            start_scatters(j1, 1)

            @pl.when(jj + 1 < half)
            def _():
                wait_scatters(0)
                start_gathers(j0 + 2, 0)

            return carry

        lax.fori_loop(0, half, body, 0)
        wait_scatters(0)
        wait_scatters(1)
        plsc.subcore_barrier()
        pltpu.sync_copy(sh_au.at[pl.ds(off, spt)],
                        aggu_out.at[c, pl.ds(off, spt)])
        pltpu.sync_copy(sh_ar.at[pl.ds(off, spt)],
                        aggr_out.at[c, pl.ds(off, spt)])

    return spmm_kernel


def _make_score_kernel(npad, srows_pad, w):
    rpt = srows_pad // NW
    spt = npad // N_TILES

    @functools.partial(
        pl.kernel,
        out_type=jax.ShapeDtypeStruct((srows_pad, LANES), jnp.float32),
        mesh=_make_mesh(),
        compiler_params=pltpu.CompilerParams(use_tc_tiling_on_sc=False, needs_layout_passes=False),
        scratch_types=[
            pltpu.VMEM((rpt, LANES), jnp.int32),
            pltpu.VMEM((rpt, LANES), jnp.int32),
            pltpu.VMEM((2, LANES, w), jnp.float32),
            pltpu.VMEM((2, LANES, w), jnp.float32),
            pltpu.VMEM((rpt, LANES), jnp.float32),
            pltpu.SemaphoreType.DMA((2,)),
            pltpu.SemaphoreType.DMA((2,)),
        ],
    )
    def score_kernel(se3, nu, nr, out,
                     aidx, bidx, arow, brow, outbuf,
                     sem_a, sem_b):
        c = lax.axis_index("c")
        s = lax.axis_index("s")
        wid = c * N_TILES + s
        pltpu.sync_copy(se3.at[0, pl.ds(wid * rpt, rpt)], aidx)
        pltpu.sync_copy(se3.at[1, pl.ds(wid * rpt, rpt)], bidx)

        lane_iota = lax.iota(jnp.int32, 16)

        def start_gathers(j, p):
            pltpu.async_copy(nu.at[aidx.at[j]], arow.at[p], sem_a.at[p])
            pltpu.async_copy(nr.at[bidx.at[j]], brow.at[p], sem_b.at[p])

        def wait_gathers(p):
            pltpu.make_async_copy(nu.at[aidx.at[0]], arow.at[p],
                                  sem_a.at[p]).wait()
            pltpu.make_async_copy(nr.at[bidx.at[0]], brow.at[p],
                                  sem_b.at[p]).wait()

        def compute(j, p):
            def grp_body(g, gcarry):
                rows = g * 16 + lane_iota
                accs = [jnp.zeros((16,), jnp.float32) for _ in range(4)]
                for col in range(0, w, 4):
                    for q in range(4):
                        cvec = jnp.full((16,), col + q, jnp.int32)
                        accs[q] = accs[q] + (
                            plsc.load_gather(arow.at[p], [rows, cvec]) *
                            plsc.load_gather(brow.at[p], [rows, cvec]))
                outbuf[j, pl.ds(g * 16, 16)] = ((accs[0] + accs[1]) +
                                                (accs[2] + accs[3]))
                return gcarry

            lax.fori_loop(0, LANES // 16, grp_body, 0)

        half = rpt // 2
        start_gathers(0, 0)

        def body(jj, carry):
            j0 = 2 * jj
            j1 = j0 + 1
            start_gathers(j1, 1)
            wait_gathers(0)
            compute(j0, 0)

            @pl.when(jj + 1 < half)
            def _():
                start_gathers(j0 + 2, 0)

            wait_gathers(1)
            compute(j1, 1)
            return carry

        lax.fori_loop(0, half, body, 0)
        pltpu.sync_copy(outbuf, out.at[pl.ds(wid * rpt, rpt)])

    return score_kernel


def kernel(user_feat, repo_feat, edge_index_ur, pos_edge_index,
           neg_edge_index, W_user, b_user, W_repo, b_repo,
           W_h_ur, b_h_ur, W_h_ru, b_h_ru,
           W_o_ur, b_o_ur, W_o_ru, b_o_ru):
    n_user, d_in = user_feat.shape
    n_repo = repo_feat.shape[0]
    n = max(n_user, n_repo)
    npad = _cdiv(n, LANES) * LANES
    if npad == n:
        npad += LANES
    d_hid = W_h_ur.shape[1]
    d_out = W_o_ur.shape[1]
    ep = pos_edge_index.shape[1]
    e = edge_index_ur.shape[1]

    npad_rows = npad - n

    def pad_edges(ei, rows_target):
        total = rows_target * LANES
        extra = total - ei.shape[1]
        padv = (n + (jnp.arange(extra, dtype=jnp.int32) % npad_rows))
        pad_block = jnp.stack([padv, padv])
        return jnp.concatenate([ei.astype(jnp.int32), pad_block],
                               axis=1).reshape(2, rows_target, LANES)

    erows_pad = _cdiv(_cdiv(e, LANES), NW * 8) * (NW * 8)
    e3 = pad_edges(edge_index_ur, erows_pad)

    sidx = jnp.concatenate([pos_edge_index, neg_edge_index], axis=1)
    srows_pad = _cdiv(_cdiv(2 * ep, LANES), NW * 8) * (NW * 8)
    se3 = pad_edges(sidx, srows_pad)

    uf = jnp.pad(user_feat, ((0, npad - n_user), (0, 0)))
    rf = jnp.pad(repo_feat, ((0, npad - n_repo), (0, 0)))
    z16 = jnp.zeros((npad, 16), jnp.float32)
    zw = jnp.zeros((npad, d_hid), jnp.float32)
    ones16 = jnp.ones((LANES, 16), jnp.float32)
    bu = b_user.reshape(1, -1)
    br = b_repo.reshape(1, -1)
    bhru = b_h_ru.reshape(1, -1)
    bhur = b_h_ur.reshape(1, -1)
    boru = b_o_ru.reshape(1, -1)
    bour = b_o_ur.reshape(1, -1)

    degu2, degr2 = _make_deg_kernel(npad, erows_pad)(e3, z16, ones16)

    xu, xr = _tc_call(functools.partial(_tc_embed_body, n),
                      [(npad, d_hid), (npad, d_hid)],
                      uf, rf, W_user, bu, W_repo, br, W_h_ur, W_h_ru,
                      degu2, degr2)

    spmm1 = _make_spmm_kernel(npad, erows_pad, d_hid)
    au2, ar2 = spmm1(e3, xu, xr, zw)

    yu, yr = _tc_call(functools.partial(_tc_mid_body, n),
                      [(npad, d_out), (npad, d_out)],
                      au2, ar2, degu2, degr2, bhru, bhur, W_o_ur, W_o_ru)

    spmm2 = _make_spmm_kernel(npad, erows_pad, d_out)
    zw2 = zw if d_out == d_hid else jnp.zeros((npad, d_out), jnp.float32)
    a2u2, a2r2 = spmm2(e3, yu, yr, zw2)

    nu, nr = _tc_call(functools.partial(_tc_final_body, n),
                      [(npad, d_out), (npad, d_out)],
                      a2u2, a2r2, degu2, degr2, boru, bour)

    scores = _make_score_kernel(npad, srows_pad, d_out)(se3, nu, nr)
    flat = scores.reshape(-1)
    return flat[:ep], flat[ep:2 * ep]
